# baseline (speedup 1.0000x reference)
import functools

import jax
import jax.numpy as jnp
from jax import lax
from jax.experimental import pallas as pl
from jax.experimental.pallas import tpu as pltpu
from jax.experimental.pallas import tpu_sc as plsc

T = 512
H = 2048
E = 16
TOP_K = 2
N_GROUP = 4
TOPK_GROUP = 2
I = 1408
SCALE = 2.5

BLK = 128
NT = 24
NT4 = NT + 4
NR = NT * BLK
NI = I // 128
NEG = -1e30


def _silu(x):
    return x * jax.nn.sigmoid(x)


def _router_body(x_ref, gw_ref, bias_ref, w0_ref, w1_ref, r0_ref, r1_ref,
                 tok_ref, eot_ref, act_ref):
    x = x_ref[...]
    gw = gw_ref[...]
    logits = lax.dot_general(x, gw, (((1,), (1,)), ((), ())),
                             preferred_element_type=jnp.float32)
    scores = jax.nn.sigmoid(logits)
    swb = scores + bias_ref[...]

    eidx = lax.broadcasted_iota(jnp.int32, (T, E), 1)
    gid = eidx // (E // N_GROUP)

    gs_full = jnp.zeros((T, E), jnp.float32)
    for g in range(N_GROUP):
        mg = gid == g
        vg = jnp.where(mg, swb, NEG)
        m1 = jnp.max(vg, axis=1, keepdims=True)
        i1 = jnp.min(jnp.where(mg & (swb == m1), eidx, 999), axis=1,
                     keepdims=True)
        m2 = jnp.max(jnp.where(mg & (eidx != i1), swb, NEG), axis=1,
                     keepdims=True)
        gs_full = gs_full + jnp.where(mg, m1 + m2, 0.0)

    gm1 = jnp.max(gs_full, axis=1, keepdims=True)
    g1 = jnp.min(jnp.where(gs_full == gm1, gid, 999), axis=1, keepdims=True)
    gm2 = jnp.max(jnp.where(gid != g1, gs_full, NEG), axis=1, keepdims=True)
    g2 = jnp.min(jnp.where((gid != g1) & (gs_full == gm2), gid, 999),
                 axis=1, keepdims=True)
    gmask = (gid == g1) | (gid == g2)
    masked = jnp.where(gmask, swb, 0.0)

    v1 = jnp.max(masked, axis=1, keepdims=True)
    e1 = jnp.min(jnp.where(masked == v1, eidx, 999), axis=1, keepdims=True)
    v2 = jnp.max(jnp.where(eidx != e1, masked, NEG), axis=1, keepdims=True)
    e2 = jnp.min(jnp.where((eidx != e1) & (masked == v2), eidx, 999),
                 axis=1, keepdims=True)
    newmask = (eidx == e1) | (eidx == e2)
    sm = jnp.where(newmask, scores, 0.0)
    sn = sm / (jnp.sum(sm, axis=1, keepdims=True) + 1e-20) * SCALE
    w0_ref[...] = jnp.sum(jnp.where(eidx == e1, sn, 0.0), axis=1,
                          keepdims=True)
    w1_ref[...] = jnp.sum(jnp.where(eidx == e2, sn, 0.0), axis=1,
                          keepdims=True)

    oh = (eidx == e1).astype(jnp.float32) + (eidx == e2).astype(jnp.float32)
    ir = lax.broadcasted_iota(jnp.int32, (T, T), 0)
    ic = lax.broadcasted_iota(jnp.int32, (T, T), 1)
    tri = (ir >= ic).astype(jnp.float32)
    cum = lax.dot_general(tri, oh, (((1,), (0,)), ((), ())),
                          preferred_element_type=jnp.float32)
    excl = cum - oh
    counts = cum[T - 1:T, :]
    counts_i = counts.astype(jnp.int32)
    tiles_e = (counts_i + (BLK - 1)) // BLK
    li = lax.broadcasted_iota(jnp.int32, (E, E), 0)
    lj = lax.broadcasted_iota(jnp.int32, (E, E), 1)
    ltm = (li < lj).astype(jnp.float32)
    tile_off = lax.dot_general(tiles_e.astype(jnp.float32), ltm,
                               (((1,), (0,)), ((), ())),
                               preferred_element_type=jnp.float32)
    tile_off_i = tile_off.astype(jnp.int32)
    pad_off = tile_off_i * BLK
    pad_b = jnp.broadcast_to(pad_off, (T, E))
    rw0 = jnp.sum(jnp.where(eidx == e1, excl, 0.0), axis=1, keepdims=True)
    rw1 = jnp.sum(jnp.where(eidx == e2, excl, 0.0), axis=1, keepdims=True)
    po0 = jnp.sum(jnp.where(eidx == e1, pad_b, 0), axis=1, keepdims=True)
    po1 = jnp.sum(jnp.where(eidx == e2, pad_b, 0), axis=1, keepdims=True)
    r0 = po0 + rw0.astype(jnp.int32)
    r1 = po1 + rw1.astype(jnp.int32)
    r0_ref[...] = r0
    r1_ref[...] = r1

    sl = lax.broadcasted_iota(jnp.int32, (T, NR), 1)
    hit = (sl == r0) | (sl == r1)
    tid = lax.broadcasted_iota(jnp.int32, (T, NR), 0)
    tok_ref[...] = jnp.sum(jnp.where(hit, tid, 0), axis=0, keepdims=True)

    total = jnp.sum(tiles_e, axis=1, keepdims=True)
    ti = lax.broadcasted_iota(jnp.int32, (NT4, E), 0)
    te = lax.broadcasted_iota(jnp.int32, (NT4, E), 1)
    toff = jnp.broadcast_to(tile_off_i, (NT4, E))
    tlen = jnp.broadcast_to(tiles_e, (NT4, E))
    owns = (ti >= toff) & (ti < toff + tlen)
    eot = jnp.sum(jnp.where(owns, te, 0), axis=1, keepdims=True)
    last_e = jnp.max(jnp.where(counts_i > 0,
                               lax.broadcasted_iota(jnp.int32, (1, E), 1),
                               0), axis=1, keepdims=True)
    ti1 = ti[:, :1]
    is_act = ti1 < total
    is_sh = ti1 >= NT
    eot_ref[...] = jnp.where(is_act, eot, last_e)
    act_ref[...] = jnp.where(is_sh, 2, jnp.where(is_act, 1, 0))


def _router(hidden, gate_weight, bias2d):
    return pl.pallas_call(
        _router_body,
        out_shape=[
            jax.ShapeDtypeStruct((T, 1), jnp.float32),
            jax.ShapeDtypeStruct((T, 1), jnp.float32),
            jax.ShapeDtypeStruct((T, 1), jnp.int32),
            jax.ShapeDtypeStruct((T, 1), jnp.int32),
            jax.ShapeDtypeStruct((1, NR), jnp.int32),
            jax.ShapeDtypeStruct((NT4, 1), jnp.int32),
            jax.ShapeDtypeStruct((NT4, 1), jnp.int32),
        ],
    )(hidden, gate_weight, bias2d)


def _sh_body(hid_ref, swg_ref, swu_ref, swd_ref, o_ref, hbf_s, y_acc):
    i = pl.program_id(0)
    j = pl.program_id(1)

    @pl.when((i == 0) & (j == 0))
    def _():
        hbf_s[...] = hid_ref[...].astype(jnp.bfloat16)

    x = hbf_s[pl.ds(i * BLK, BLK), :]
    a = _bdot(x, swg_ref[...].astype(jnp.bfloat16))
    b = _bdot(x, swu_ref[...].astype(jnp.bfloat16))
    h = (_silu(a) * b).astype(jnp.bfloat16)
    c = _bdot(h, swd_ref[...].astype(jnp.bfloat16))

    @pl.when(j == 0)
    def _():
        y_acc[...] = c

    @pl.when(j != 0)
    def _():
        y_acc[...] += c

    @pl.when(j == NI - 1)
    def _():
        o_ref[...] = y_acc[...]


def _shared(hidden, sw_gate, sw_up, sw_down):
    return pl.pallas_call(
        _sh_body,
        grid=(T // BLK, NI),
        in_specs=[
            pl.BlockSpec((T, H), lambda i, j: (0, 0)),
            pl.BlockSpec((H, 128), lambda i, j: (0, j)),
            pl.BlockSpec((H, 128), lambda i, j: (0, j)),
            pl.BlockSpec((128, H), lambda i, j: (j, 0)),
        ],
        out_specs=pl.BlockSpec((BLK, H), lambda i, j: (i, 0)),
        out_shape=jax.ShapeDtypeStruct((T, H), jnp.float32),
        scratch_shapes=[
            pltpu.VMEM((T, H), jnp.bfloat16),
            pltpu.VMEM((BLK, H), jnp.float32),
        ],
    )(hidden, sw_gate, sw_up, sw_down)


SC_NC = 2
SC_NS = 16
SC_ROWS = NR // (SC_NC * SC_NS)
SC_CH = 32


def _sc_gather_body(tok_hbm, hid_hbm, out_hbm, idx_v, rows_v, sem):
    wid = lax.axis_index("s") * SC_NC + lax.axis_index("c")
    base = wid * SC_ROWS
    for c in range(SC_ROWS // SC_CH):
        b = base + c * SC_CH
        pltpu.sync_copy(tok_hbm.at[pl.ds(b, SC_CH)], idx_v)
        pltpu.async_copy(hid_hbm.at[idx_v], rows_v, sem).wait()
        pltpu.sync_copy(rows_v, out_hbm.at[pl.ds(b, SC_CH)])


def _sc_gather(tok_flat, hidden):
    mesh = plsc.VectorSubcoreMesh(core_axis_name="c", subcore_axis_name="s")
    f = functools.partial(
        pl.kernel,
        mesh=mesh,
        out_type=jax.ShapeDtypeStruct((NR, H), jnp.float32),
        scratch_types=[
            pltpu.VMEM((SC_CH,), jnp.int32),
            pltpu.VMEM((SC_CH, H), jnp.float32),
            pltpu.SemaphoreType.DMA,
        ],
    )(_sc_gather_body)
    return f(tok_flat, hidden)


def _bdot(a, b):
    return lax.dot_general(a, b, (((1,), (0,)), ((), ())),
                           preferred_element_type=jnp.float32)


def _gu_body(eot_ref, act_ref, xg_ref, wg_ref, wu_ref, h_ref):
    i = pl.program_id(0)

    @pl.when(act_ref[i] == 1)
    def _():
        x = xg_ref[...].astype(jnp.bfloat16)
        a = _bdot(x, wg_ref[0].astype(jnp.bfloat16))
        b = _bdot(x, wu_ref[0].astype(jnp.bfloat16))
        h_ref[...] = (_silu(a) * b).astype(jnp.bfloat16)


def _gate_up(x_g, w_gate, w_up, eot, act):
    grid_spec = pltpu.PrefetchScalarGridSpec(
        num_scalar_prefetch=2,
        grid=(NT,),
        in_specs=[
            pl.BlockSpec((BLK, H), lambda i, eot, act: (i, 0)),
            pl.BlockSpec((1, H, I), lambda i, eot, act: (eot[i], 0, 0)),
            pl.BlockSpec((1, H, I), lambda i, eot, act: (eot[i], 0, 0)),
        ],
        out_specs=pl.BlockSpec((BLK, I), lambda i, eot, act: (i, 0)),
    )
    return pl.pallas_call(
        _gu_body,
        grid_spec=grid_spec,
        out_shape=jax.ShapeDtypeStruct((NR, I), jnp.bfloat16),
    )(eot, act, x_g, w_gate, w_up)


def _dn_body(eot_ref, act_ref, h_ref, wd_ref, sh_ref, r0_ref, r1_ref,
             w0_ref, w1_ref, o_ref):
    i = pl.program_id(0)

    @pl.when(act_ref[i] == 1)
    def _():
        y = _bdot(h_ref[...], wd_ref[0].astype(jnp.bfloat16))
        sl = lax.broadcasted_iota(jnp.int32, (T, BLK), 1) + i * BLK
        m = (jnp.where(r0_ref[...] == sl, w0_ref[...], 0.0) +
             jnp.where(r1_ref[...] == sl, w1_ref[...], 0.0)).astype(
                 jnp.bfloat16)
        contrib = _bdot(m, y.astype(jnp.bfloat16))

        @pl.when(i == 0)
        def _():
            o_ref[...] = sh_ref[...] + contrib

        @pl.when(i != 0)
        def _():
            o_ref[...] += contrib


def _down_combine(h_out, w_down, shared_y, r0, r1, w0, w1, eot, act):
    grid_spec = pltpu.PrefetchScalarGridSpec(
        num_scalar_prefetch=2,
        grid=(NT,),
        in_specs=[
            pl.BlockSpec((BLK, I), lambda i, eot, act: (i, 0)),
            pl.BlockSpec((1, I, H), lambda i, eot, act: (eot[i], 0, 0)),
            pl.BlockSpec((T, H), lambda i, eot, act: (0, 0)),
            pl.BlockSpec((T, 1), lambda i, eot, act: (0, 0)),
            pl.BlockSpec((T, 1), lambda i, eot, act: (0, 0)),
            pl.BlockSpec((T, 1), lambda i, eot, act: (0, 0)),
            pl.BlockSpec((T, 1), lambda i, eot, act: (0, 0)),
        ],
        out_specs=pl.BlockSpec((T, H), lambda i, eot, act: (0, 0)),
    )
    return pl.pallas_call(
        _dn_body,
        grid_spec=grid_spec,
        out_shape=jax.ShapeDtypeStruct((T, H), jnp.float32),
    )(eot, act, h_out, w_down, shared_y, r0, r1, w0, w1)


def kernel(hidden_states, gate_weight, e_score_correction_bias, w_gate,
           w_up, w_down, sw_gate, sw_up, sw_down):
    bias2d = e_score_correction_bias.reshape(1, E)
    w0, w1, r0, r1, tok, eot, act = _router(hidden_states, gate_weight,
                                            bias2d)
    eot_v = eot.reshape(NT4)[:NT]
    act_v = act.reshape(NT4)[:NT]
    x_g = _sc_gather(tok.reshape(NR), hidden_states)
    shared_y = _shared(hidden_states, sw_gate, sw_up, sw_down)
    h_out = _gate_up(x_g, w_gate, w_up, eot_v, act_v)
    return _down_combine(h_out, w_down, shared_y, r0, r1, w0, w1, eot_v,
                         act_v)


# speedup vs baseline: 1.3494x; 1.3494x over previous
import jax
import jax.numpy as jnp
from jax import lax
from jax.experimental import pallas as pl
from jax.experimental.pallas import tpu as pltpu

T = 512
H = 2048
E = 16
TOP_K = 2
N_GROUP = 4
TOPK_GROUP = 2
I = 1408
SCALE = 2.5

BLK = 128
NT = 24
NT4 = NT + 4
NR = NT * BLK
NI = I // 128
NEG = -1e30


def _silu(x):
    return x * jax.nn.sigmoid(x)


def _router_body(x_ref, gw_ref, bias_ref, w0_ref, w1_ref, r0_ref, r1_ref,
                 tok_ref, eot_ref, act_ref):
    x = x_ref[...]
    gw = gw_ref[...]
    logits = lax.dot_general(x, gw, (((1,), (1,)), ((), ())),
                             preferred_element_type=jnp.float32)
    scores = jax.nn.sigmoid(logits)
    swb = scores + bias_ref[...]

    eidx = lax.broadcasted_iota(jnp.int32, (T, E), 1)
    gid = eidx // (E // N_GROUP)

    gs_full = jnp.zeros((T, E), jnp.float32)
    for g in range(N_GROUP):
        mg = gid == g
        vg = jnp.where(mg, swb, NEG)
        m1 = jnp.max(vg, axis=1, keepdims=True)
        i1 = jnp.min(jnp.where(mg & (swb == m1), eidx, 999), axis=1,
                     keepdims=True)
        m2 = jnp.max(jnp.where(mg & (eidx != i1), swb, NEG), axis=1,
                     keepdims=True)
        gs_full = gs_full + jnp.where(mg, m1 + m2, 0.0)

    gm1 = jnp.max(gs_full, axis=1, keepdims=True)
    g1 = jnp.min(jnp.where(gs_full == gm1, gid, 999), axis=1, keepdims=True)
    gm2 = jnp.max(jnp.where(gid != g1, gs_full, NEG), axis=1, keepdims=True)
    g2 = jnp.min(jnp.where((gid != g1) & (gs_full == gm2), gid, 999),
                 axis=1, keepdims=True)
    gmask = (gid == g1) | (gid == g2)
    masked = jnp.where(gmask, swb, 0.0)

    v1 = jnp.max(masked, axis=1, keepdims=True)
    e1 = jnp.min(jnp.where(masked == v1, eidx, 999), axis=1, keepdims=True)
    v2 = jnp.max(jnp.where(eidx != e1, masked, NEG), axis=1, keepdims=True)
    e2 = jnp.min(jnp.where((eidx != e1) & (masked == v2), eidx, 999),
                 axis=1, keepdims=True)
    newmask = (eidx == e1) | (eidx == e2)
    sm = jnp.where(newmask, scores, 0.0)
    sn = sm / (jnp.sum(sm, axis=1, keepdims=True) + 1e-20) * SCALE
    w0_ref[...] = jnp.sum(jnp.where(eidx == e1, sn, 0.0), axis=1,
                          keepdims=True)
    w1_ref[...] = jnp.sum(jnp.where(eidx == e2, sn, 0.0), axis=1,
                          keepdims=True)

    oh = (eidx == e1).astype(jnp.float32) + (eidx == e2).astype(jnp.float32)
    ir = lax.broadcasted_iota(jnp.int32, (T, T), 0)
    ic = lax.broadcasted_iota(jnp.int32, (T, T), 1)
    tri = (ir >= ic).astype(jnp.float32)
    cum = lax.dot_general(tri, oh, (((1,), (0,)), ((), ())),
                          preferred_element_type=jnp.float32)
    excl = cum - oh
    counts = cum[T - 1:T, :]
    counts_i = counts.astype(jnp.int32)
    tiles_e = (counts_i + (BLK - 1)) // BLK
    li = lax.broadcasted_iota(jnp.int32, (E, E), 0)
    lj = lax.broadcasted_iota(jnp.int32, (E, E), 1)
    ltm = (li < lj).astype(jnp.float32)
    tile_off = lax.dot_general(tiles_e.astype(jnp.float32), ltm,
                               (((1,), (0,)), ((), ())),
                               preferred_element_type=jnp.float32)
    tile_off_i = tile_off.astype(jnp.int32)
    pad_off = tile_off_i * BLK
    pad_b = jnp.broadcast_to(pad_off, (T, E))
    rw0 = jnp.sum(jnp.where(eidx == e1, excl, 0.0), axis=1, keepdims=True)
    rw1 = jnp.sum(jnp.where(eidx == e2, excl, 0.0), axis=1, keepdims=True)
    po0 = jnp.sum(jnp.where(eidx == e1, pad_b, 0), axis=1, keepdims=True)
    po1 = jnp.sum(jnp.where(eidx == e2, pad_b, 0), axis=1, keepdims=True)
    r0 = po0 + rw0.astype(jnp.int32)
    r1 = po1 + rw1.astype(jnp.int32)
    r0_ref[...] = r0
    r1_ref[...] = r1

    sl = lax.broadcasted_iota(jnp.int32, (T, NR), 1)
    hit = (sl == r0) | (sl == r1)
    tid = lax.broadcasted_iota(jnp.int32, (T, NR), 0)
    tok_ref[...] = jnp.sum(jnp.where(hit, tid, 0), axis=0, keepdims=True)

    total = jnp.sum(tiles_e, axis=1, keepdims=True)
    ti = lax.broadcasted_iota(jnp.int32, (NT4, E), 0)
    te = lax.broadcasted_iota(jnp.int32, (NT4, E), 1)
    toff = jnp.broadcast_to(tile_off_i, (NT4, E))
    tlen = jnp.broadcast_to(tiles_e, (NT4, E))
    owns = (ti >= toff) & (ti < toff + tlen)
    eot = jnp.sum(jnp.where(owns, te, 0), axis=1, keepdims=True)
    last_e = jnp.max(jnp.where(counts_i > 0,
                               lax.broadcasted_iota(jnp.int32, (1, E), 1),
                               0), axis=1, keepdims=True)
    ti1 = ti[:, :1]
    is_act = ti1 < total
    is_sh = ti1 >= NT
    eot_ref[...] = jnp.where(is_act, eot, last_e)
    act_ref[...] = jnp.where(is_sh, 2, jnp.where(is_act, 1, 0))


def _router(hidden, gate_weight, bias2d):
    return pl.pallas_call(
        _router_body,
        out_shape=[
            jax.ShapeDtypeStruct((T, 1), jnp.float32),
            jax.ShapeDtypeStruct((T, 1), jnp.float32),
            jax.ShapeDtypeStruct((T, 1), jnp.int32),
            jax.ShapeDtypeStruct((T, 1), jnp.int32),
            jax.ShapeDtypeStruct((1, NR), jnp.int32),
            jax.ShapeDtypeStruct((NT4, 1), jnp.int32),
            jax.ShapeDtypeStruct((NT4, 1), jnp.int32),
        ],
    )(hidden, gate_weight, bias2d)


def _sh_body(hid_ref, swg_ref, swu_ref, swd_ref, rt_ref, o_ref, hbf_s,
             y_acc):
    i = pl.program_id(0)
    j = pl.program_id(1)

    @pl.when((i == 0) & (j == 0))
    def _():
        hbf_s[...] = hid_ref[...].astype(jnp.bfloat16)

    x = hbf_s[pl.ds(i * BLK, BLK), :]
    a = _bdot(x, swg_ref[...].astype(jnp.bfloat16))
    b = _bdot(x, swu_ref[...].astype(jnp.bfloat16))
    h = (_silu(a) * b).astype(jnp.bfloat16)
    c = _bdot(h, swd_ref[...].astype(jnp.bfloat16))

    @pl.when(j == 0)
    def _():
        y_acc[...] = c

    @pl.when(j != 0)
    def _():
        y_acc[...] += c

    @pl.when(j == NI - 1)
    def _():
        o_ref[...] = y_acc[...] + rt_ref[...]


def _shared(hidden, sw_gate, sw_up, sw_down, routed):
    return pl.pallas_call(
        _sh_body,
        grid=(T // BLK, NI),
        in_specs=[
            pl.BlockSpec((T, H), lambda i, j: (0, 0)),
            pl.BlockSpec((H, 128), lambda i, j: (0, j)),
            pl.BlockSpec((H, 128), lambda i, j: (0, j)),
            pl.BlockSpec((128, H), lambda i, j: (j, 0)),
            pl.BlockSpec((BLK, H), lambda i, j: (i, 0)),
        ],
        out_specs=pl.BlockSpec((BLK, H), lambda i, j: (i, 0)),
        out_shape=jax.ShapeDtypeStruct((T, H), jnp.float32),
        scratch_shapes=[
            pltpu.VMEM((T, H), jnp.bfloat16),
            pltpu.VMEM((BLK, H), jnp.float32),
        ],
    )(hidden, sw_gate, sw_up, sw_down, routed)


KH = H // 2


def _bdot(a, b):
    return lax.dot_general(a, b, (((1,), (0,)), ((), ())),
                           preferred_element_type=jnp.float32)


def _m2_body(eot_ref, act_ref, tok_ref, r0_ref, r1_ref, w0_ref, w1_ref,
             hid_ref, wg_ref, wu_ref, wd_ref, o_ref,
             hbf_s, x_s, a_s, b_s, wd_s):
    i = pl.program_id(0)
    k = pl.program_id(1)

    @pl.when((i == 0) & (k == 0))
    def _():
        hbf_s[...] = hid_ref[...].astype(jnp.bfloat16)

    @pl.when(act_ref[i] == 1)
    def _():
        @pl.when(k == 0)
        def _():
            tok = tok_ref[0, 0, :]
            ohm = (tok[:, None] ==
                   lax.broadcasted_iota(jnp.int32, (BLK, T), 1)).astype(
                       jnp.bfloat16)
            x_s[...] = _bdot(ohm, hbf_s[...]).astype(jnp.bfloat16)
            a_s[...] = _bdot(x_s[:, :KH], wg_ref[0].astype(jnp.bfloat16))
            b_s[...] = _bdot(x_s[:, :KH], wu_ref[0].astype(jnp.bfloat16))
            wd_s[...] = wd_ref[0].astype(jnp.bfloat16)

        @pl.when(k == 1)
        def _():
            a = a_s[...] + _bdot(x_s[:, KH:],
                                 wg_ref[0].astype(jnp.bfloat16))
            b = b_s[...] + _bdot(x_s[:, KH:],
                                 wu_ref[0].astype(jnp.bfloat16))
            h = (_silu(a) * b).astype(jnp.bfloat16)
            y_lo = _bdot(h, wd_s[...]).astype(jnp.bfloat16)
            y_hi = _bdot(h, wd_ref[0].astype(jnp.bfloat16)).astype(
                jnp.bfloat16)
            sl = lax.broadcasted_iota(jnp.int32, (T, BLK), 1) + i * BLK
            m = (jnp.where(r0_ref[...] == sl, w0_ref[...], 0.0) +
                 jnp.where(r1_ref[...] == sl, w1_ref[...], 0.0)).astype(
                     jnp.bfloat16)
            c_lo = _bdot(m, y_lo)
            c_hi = _bdot(m, y_hi)

            @pl.when(i == 0)
            def _():
                o_ref[:, :KH] = c_lo
                o_ref[:, KH:] = c_hi

            @pl.when(i != 0)
            def _():
                o_ref[:, :KH] += c_lo
                o_ref[:, KH:] += c_hi


def _moe2(tok3d, hidden, w_gate, w_up, w_down, r0, r1, w0, w1, eot, act):
    grid_spec = pltpu.PrefetchScalarGridSpec(
        num_scalar_prefetch=2,
        grid=(NT, 2),
        in_specs=[
            pl.BlockSpec((1, 1, BLK),
                         lambda i, k, eot, act: (i, 0, 0)),
            pl.BlockSpec((T, 1), lambda i, k, eot, act: (0, 0)),
            pl.BlockSpec((T, 1), lambda i, k, eot, act: (0, 0)),
            pl.BlockSpec((T, 1), lambda i, k, eot, act: (0, 0)),
            pl.BlockSpec((T, 1), lambda i, k, eot, act: (0, 0)),
            pl.BlockSpec((T, H), lambda i, k, eot, act: (0, 0)),
            pl.BlockSpec((1, KH, I),
                         lambda i, k, eot, act: (eot[i], k, 0)),
            pl.BlockSpec((1, KH, I),
                         lambda i, k, eot, act: (eot[i], k, 0)),
            pl.BlockSpec((1, I, KH),
                         lambda i, k, eot, act: (eot[i], 0, k)),
        ],
        out_specs=pl.BlockSpec((T, H), lambda i, k, eot, act: (0, 0)),
        scratch_shapes=[
            pltpu.VMEM((T, H), jnp.bfloat16),
            pltpu.VMEM((BLK, H), jnp.bfloat16),
            pltpu.VMEM((BLK, I), jnp.float32),
            pltpu.VMEM((BLK, I), jnp.float32),
            pltpu.VMEM((I, KH), jnp.bfloat16),
        ],
    )
    return pl.pallas_call(
        _m2_body,
        grid_spec=grid_spec,
        out_shape=jax.ShapeDtypeStruct((T, H), jnp.float32),
    )(eot, act, tok3d, r0, r1, w0, w1, hidden, w_gate, w_up, w_down)


def kernel(hidden_states, gate_weight, e_score_correction_bias, w_gate,
           w_up, w_down, sw_gate, sw_up, sw_down):
    bias2d = e_score_correction_bias.reshape(1, E)
    w0, w1, r0, r1, tok, eot, act = _router(hidden_states, gate_weight,
                                            bias2d)
    tok3d = tok.reshape(NT, 1, BLK)
    eot_v = eot.reshape(NT4)[:NT]
    act_v = act.reshape(NT4)[:NT]
    routed = _moe2(tok3d, hidden_states, w_gate, w_up, w_down, r0, r1,
                   w0, w1, eot_v, act_v)
    return _shared(hidden_states, sw_gate, sw_up, sw_down, routed)


# speedup vs baseline: 1.6863x; 1.2497x over previous
import jax
import jax.numpy as jnp
from jax import lax
from jax.experimental import pallas as pl
from jax.experimental.pallas import tpu as pltpu

T = 512
H = 2048
E = 16
TOP_K = 2
N_GROUP = 4
TOPK_GROUP = 2
I = 1408
SCALE = 2.5

BLK = 128
NT = 24
NT4 = NT + 4
NR = NT * BLK
NI = I // 128
NEG = -1e30


def _silu(x):
    return x * jax.nn.sigmoid(x)


def _router_body(x_ref, gw_ref, bias_ref, w0_ref, w1_ref, r0_ref, r1_ref,
                 tok_ref, eot_ref, act_ref):
    x = x_ref[...]
    gw = gw_ref[...]
    logits = lax.dot_general(x, gw, (((1,), (1,)), ((), ())),
                             preferred_element_type=jnp.float32)
    scores = jax.nn.sigmoid(logits)
    swb = scores + bias_ref[...]

    eidx = lax.broadcasted_iota(jnp.int32, (T, E), 1)
    gid = eidx // (E // N_GROUP)

    gs_full = jnp.zeros((T, E), jnp.float32)
    for g in range(N_GROUP):
        mg = gid == g
        vg = jnp.where(mg, swb, NEG)
        m1 = jnp.max(vg, axis=1, keepdims=True)
        i1 = jnp.min(jnp.where(mg & (swb == m1), eidx, 999), axis=1,
                     keepdims=True)
        m2 = jnp.max(jnp.where(mg & (eidx != i1), swb, NEG), axis=1,
                     keepdims=True)
        gs_full = gs_full + jnp.where(mg, m1 + m2, 0.0)

    gm1 = jnp.max(gs_full, axis=1, keepdims=True)
    g1 = jnp.min(jnp.where(gs_full == gm1, gid, 999), axis=1, keepdims=True)
    gm2 = jnp.max(jnp.where(gid != g1, gs_full, NEG), axis=1, keepdims=True)
    g2 = jnp.min(jnp.where((gid != g1) & (gs_full == gm2), gid, 999),
                 axis=1, keepdims=True)
    gmask = (gid == g1) | (gid == g2)
    masked = jnp.where(gmask, swb, 0.0)

    v1 = jnp.max(masked, axis=1, keepdims=True)
    e1 = jnp.min(jnp.where(masked == v1, eidx, 999), axis=1, keepdims=True)
    v2 = jnp.max(jnp.where(eidx != e1, masked, NEG), axis=1, keepdims=True)
    e2 = jnp.min(jnp.where((eidx != e1) & (masked == v2), eidx, 999),
                 axis=1, keepdims=True)
    newmask = (eidx == e1) | (eidx == e2)
    sm = jnp.where(newmask, scores, 0.0)
    sn = sm / (jnp.sum(sm, axis=1, keepdims=True) + 1e-20) * SCALE
    w0_ref[...] = jnp.sum(jnp.where(eidx == e1, sn, 0.0), axis=1,
                          keepdims=True)
    w1_ref[...] = jnp.sum(jnp.where(eidx == e2, sn, 0.0), axis=1,
                          keepdims=True)

    oh = (eidx == e1).astype(jnp.float32) + (eidx == e2).astype(jnp.float32)
    ir = lax.broadcasted_iota(jnp.int32, (T, T), 0)
    ic = lax.broadcasted_iota(jnp.int32, (T, T), 1)
    tri = (ir >= ic).astype(jnp.float32)
    cum = lax.dot_general(tri, oh, (((1,), (0,)), ((), ())),
                          preferred_element_type=jnp.float32)
    excl = cum - oh
    counts = cum[T - 1:T, :]
    counts_i = counts.astype(jnp.int32)
    tiles_e = (counts_i + (BLK - 1)) // BLK
    li = lax.broadcasted_iota(jnp.int32, (E, E), 0)
    lj = lax.broadcasted_iota(jnp.int32, (E, E), 1)
    ltm = (li < lj).astype(jnp.float32)
    tile_off = lax.dot_general(tiles_e.astype(jnp.float32), ltm,
                               (((1,), (0,)), ((), ())),
                               preferred_element_type=jnp.float32)
    tile_off_i = tile_off.astype(jnp.int32)
    pad_off = tile_off_i * BLK
    pad_b = jnp.broadcast_to(pad_off, (T, E))
    rw0 = jnp.sum(jnp.where(eidx == e1, excl, 0.0), axis=1, keepdims=True)
    rw1 = jnp.sum(jnp.where(eidx == e2, excl, 0.0), axis=1, keepdims=True)
    po0 = jnp.sum(jnp.where(eidx == e1, pad_b, 0), axis=1, keepdims=True)
    po1 = jnp.sum(jnp.where(eidx == e2, pad_b, 0), axis=1, keepdims=True)
    r0 = po0 + rw0.astype(jnp.int32)
    r1 = po1 + rw1.astype(jnp.int32)
    r0_ref[...] = r0
    r1_ref[...] = r1

    sl = lax.broadcasted_iota(jnp.int32, (T, NR), 1)
    hit = (sl == r0) | (sl == r1)
    tid = lax.broadcasted_iota(jnp.int32, (T, NR), 0)
    tok_ref[...] = jnp.sum(jnp.where(hit, tid, 0), axis=0, keepdims=True)

    total = jnp.sum(tiles_e, axis=1, keepdims=True)
    ti = lax.broadcasted_iota(jnp.int32, (NT4, E), 0)
    te = lax.broadcasted_iota(jnp.int32, (NT4, E), 1)
    toff = jnp.broadcast_to(tile_off_i, (NT4, E))
    tlen = jnp.broadcast_to(tiles_e, (NT4, E))
    owns = (ti >= toff) & (ti < toff + tlen)
    eot = jnp.sum(jnp.where(owns, te, 0), axis=1, keepdims=True)
    last_e = jnp.max(jnp.where(counts_i > 0,
                               lax.broadcasted_iota(jnp.int32, (1, E), 1),
                               0), axis=1, keepdims=True)
    ti1 = ti[:, :1]
    is_act = ti1 < total
    is_sh = ti1 >= NT
    eot_ref[...] = jnp.where(is_act, eot, last_e)
    act_ref[...] = jnp.where(is_sh, 2, jnp.where(is_act, 1, 0))


def _router(hidden, gate_weight, bias2d):
    return pl.pallas_call(
        _router_body,
        out_shape=[
            jax.ShapeDtypeStruct((T, 1), jnp.float32),
            jax.ShapeDtypeStruct((T, 1), jnp.float32),
            jax.ShapeDtypeStruct((T, 1), jnp.int32),
            jax.ShapeDtypeStruct((T, 1), jnp.int32),
            jax.ShapeDtypeStruct((1, NR), jnp.int32),
            jax.ShapeDtypeStruct((NT4, 1), jnp.int32),
            jax.ShapeDtypeStruct((NT4, 1), jnp.int32),
        ],
    )(hidden, gate_weight, bias2d)


def _sh_body(hid_ref, swg_ref, swu_ref, swd_ref, o_ref, hbf_s, y_acc):
    i = pl.program_id(0)
    j = pl.program_id(1)

    @pl.when((i == 0) & (j == 0))
    def _():
        hbf_s[...] = hid_ref[...].astype(jnp.bfloat16)

    x = hbf_s[pl.ds(i * BLK, BLK), :]
    a = _bdot(x, swg_ref[...].astype(jnp.bfloat16))
    b = _bdot(x, swu_ref[...].astype(jnp.bfloat16))
    h = (_silu(a) * b).astype(jnp.bfloat16)
    c = _bdot(h, swd_ref[...].astype(jnp.bfloat16))

    @pl.when(j == 0)
    def _():
        y_acc[...] = c

    @pl.when(j != 0)
    def _():
        y_acc[...] += c

    @pl.when(j == NI - 1)
    def _():
        o_ref[...] = y_acc[...]


def _shared(hidden, sw_gate, sw_up, sw_down):
    return pl.pallas_call(
        _sh_body,
        grid=(T // BLK, NI),
        in_specs=[
            pl.BlockSpec((T, H), lambda i, j: (0, 0)),
            pl.BlockSpec((H, 128), lambda i, j: (0, j)),
            pl.BlockSpec((H, 128), lambda i, j: (0, j)),
            pl.BlockSpec((128, H), lambda i, j: (j, 0)),
        ],
        out_specs=pl.BlockSpec((BLK, H), lambda i, j: (i, 0)),
        out_shape=jax.ShapeDtypeStruct((T, H), jnp.float32),
        scratch_shapes=[
            pltpu.VMEM((T, H), jnp.bfloat16),
            pltpu.VMEM((BLK, H), jnp.float32),
        ],
    )(hidden, sw_gate, sw_up, sw_down)


def _bdot(a, b):
    return lax.dot_general(a, b, (((1,), (0,)), ((), ())),
                           preferred_element_type=jnp.float32)


def _gu_body(eot_ref, act_ref, tok_ref, hid_ref, wg_ref, wu_ref, h_ref,
             hbf_s):
    i = pl.program_id(0)

    @pl.when(i == 0)
    def _():
        hbf_s[...] = hid_ref[...].astype(jnp.bfloat16)

    @pl.when(act_ref[i] == 1)
    def _():
        tok = tok_ref[0, 0, :]
        ohm = (tok[:, None] ==
               lax.broadcasted_iota(jnp.int32, (BLK, T), 1)).astype(
                   jnp.bfloat16)
        x = _bdot(ohm, hbf_s[...]).astype(jnp.bfloat16)
        a = _bdot(x, wg_ref[0].astype(jnp.bfloat16))
        b = _bdot(x, wu_ref[0].astype(jnp.bfloat16))
        h_ref[...] = (_silu(a) * b).astype(jnp.bfloat16)


def _gate_up(tok3d, hidden, w_gate, w_up, eot, act):
    grid_spec = pltpu.PrefetchScalarGridSpec(
        num_scalar_prefetch=2,
        grid=(NT,),
        in_specs=[
            pl.BlockSpec((1, 1, BLK), lambda i, eot, act: (i, 0, 0)),
            pl.BlockSpec((T, H), lambda i, eot, act: (0, 0)),
            pl.BlockSpec((1, H, I), lambda i, eot, act: (eot[i], 0, 0)),
            pl.BlockSpec((1, H, I), lambda i, eot, act: (eot[i], 0, 0)),
        ],
        out_specs=pl.BlockSpec((BLK, I), lambda i, eot, act: (i, 0)),
        scratch_shapes=[pltpu.VMEM((T, H), jnp.bfloat16)],
    )
    return pl.pallas_call(
        _gu_body,
        grid_spec=grid_spec,
        out_shape=jax.ShapeDtypeStruct((NR, I), jnp.bfloat16),
    )(eot, act, tok3d, hidden, w_gate, w_up)


def _dn_body(eot_ref, act_ref, h_ref, wd_ref, sh_ref, r0_ref, r1_ref,
             w0_ref, w1_ref, o_ref):
    i = pl.program_id(0)

    @pl.when(act_ref[i] == 1)
    def _():
        y = _bdot(h_ref[...], wd_ref[0].astype(jnp.bfloat16))
        sl = lax.broadcasted_iota(jnp.int32, (T, BLK), 1) + i * BLK
        m = (jnp.where(r0_ref[...] == sl, w0_ref[...], 0.0) +
             jnp.where(r1_ref[...] == sl, w1_ref[...], 0.0)).astype(
                 jnp.bfloat16)
        contrib = _bdot(m, y.astype(jnp.bfloat16))

        @pl.when(i == 0)
        def _():
            o_ref[...] = sh_ref[...] + contrib

        @pl.when(i != 0)
        def _():
            o_ref[...] += contrib


def _down_combine(h_out, w_down, shared_y, r0, r1, w0, w1, eot, act):
    grid_spec = pltpu.PrefetchScalarGridSpec(
        num_scalar_prefetch=2,
        grid=(NT,),
        in_specs=[
            pl.BlockSpec((BLK, I), lambda i, eot, act: (i, 0)),
            pl.BlockSpec((1, I, H), lambda i, eot, act: (eot[i], 0, 0)),
            pl.BlockSpec((T, H), lambda i, eot, act: (0, 0)),
            pl.BlockSpec((T, 1), lambda i, eot, act: (0, 0)),
            pl.BlockSpec((T, 1), lambda i, eot, act: (0, 0)),
            pl.BlockSpec((T, 1), lambda i, eot, act: (0, 0)),
            pl.BlockSpec((T, 1), lambda i, eot, act: (0, 0)),
        ],
        out_specs=pl.BlockSpec((T, H), lambda i, eot, act: (0, 0)),
    )
    return pl.pallas_call(
        _dn_body,
        grid_spec=grid_spec,
        out_shape=jax.ShapeDtypeStruct((T, H), jnp.float32),
    )(eot, act, h_out, w_down, shared_y, r0, r1, w0, w1)


def kernel(hidden_states, gate_weight, e_score_correction_bias, w_gate,
           w_up, w_down, sw_gate, sw_up, sw_down):
    bias2d = e_score_correction_bias.reshape(1, E)
    w0, w1, r0, r1, tok, eot, act = _router(hidden_states, gate_weight,
                                            bias2d)
    tok3d = tok.reshape(NT, 1, BLK)
    eot_v = eot.reshape(NT4)[:NT]
    act_v = act.reshape(NT4)[:NT]
    shared_y = _shared(hidden_states, sw_gate, sw_up, sw_down)
    h_out = _gate_up(tok3d, hidden_states, w_gate, w_up, eot_v, act_v)
    return _down_combine(h_out, w_down, shared_y, r0, r1, w0, w1, eot_v,
                         act_v)
